# baseline (speedup 1.0000x reference)
import jax
import jax.numpy as jnp
from jax import lax
from jax.experimental import pallas as pl
from jax.experimental.pallas import tpu as pltpu
from jax.experimental.pallas import tpu_sc as plsc
from functools import partial

_N_TOKENS = 8192
_DIM = 4096
_E = 64
_EPS = 1e-06
_TILE = 1024
_LANES = 16
_NC = 2
_NS = 16
_NW = _NC * _NS
_ROWS_PER_W = _N_TOKENS // _NW
_GROUPS = _ROWS_PER_W // _LANES


def _logits_kernel(x_ref, wt_ref, b_ref, out_ref):
    out_ref[...] = jnp.dot(x_ref[...], wt_ref[...],
                           preferred_element_type=jnp.float32) + b_ref[...]


def _route_kernel(logits_hbm, val_hbm, arg_hbm, ltile, vtile, atile):
    wid = lax.axis_index("s") * _NC + lax.axis_index("c")
    base = wid * _ROWS_PER_W
    pltpu.sync_copy(logits_hbm.at[pl.ds(base * _E, _ROWS_PER_W * _E)], ltile)

    def group(g, _):
        lbase = (g * _LANES + lax.iota(jnp.int32, _LANES)) * _E
        m = jnp.full((_LANES,), -jnp.inf, jnp.float32)
        a = jnp.zeros((_LANES,), jnp.int32)
        for e in range(_E):
            v = plsc.load_gather(ltile, [lbase + e])
            gt = v > m
            m = jnp.where(gt, v, m)
            a = jnp.where(gt, jnp.full((_LANES,), e, jnp.int32), a)
        s = jnp.zeros((_LANES,), jnp.float32)
        for e in range(_E):
            v = plsc.load_gather(ltile, [lbase + e])
            s = s + jnp.exp(v - m)
        vtile[pl.ds(g * _LANES, _LANES)] = 1.0 / s
        atile[pl.ds(g * _LANES, _LANES)] = a
        return ()

    lax.fori_loop(0, _GROUPS, group, ())
    pltpu.sync_copy(vtile, val_hbm.at[pl.ds(base, _ROWS_PER_W)])
    pltpu.sync_copy(atile, arg_hbm.at[pl.ds(base, _ROWS_PER_W)])


def _assemble_kernel(val_ref, arg_ref, out_ref, colsum_ref, *, n_tiles, tile,
                     capacity):
    i = pl.program_id(0)
    iota = jax.lax.broadcasted_iota(jnp.int32, (tile, _E), 1)
    masked = jnp.where(iota == arg_ref[...], val_ref[...], 0.0)

    @pl.when(i == 0)
    def _init():
        colsum_ref[...] = jnp.zeros_like(colsum_ref)

    colsum_ref[...] += jnp.sum(masked, axis=0, keepdims=True)
    out_ref[pl.ds(i * tile, tile), :] = masked

    @pl.when(i == n_tiles - 1)
    def _normalize():
        scale = capacity / (colsum_ref[...] + _EPS)
        out_ref[...] = out_ref[...] * scale


def kernel(x, W, b):
    n_tiles = _N_TOKENS // _TILE
    wt = W.T
    b2 = b.reshape(1, _E)
    capacity = float(_N_TOKENS)

    logits = pl.pallas_call(
        _logits_kernel,
        grid=(n_tiles,),
        in_specs=[
            pl.BlockSpec((_TILE, _DIM), lambda i: (i, 0)),
            pl.BlockSpec((_DIM, _E), lambda i: (0, 0)),
            pl.BlockSpec((1, _E), lambda i: (0, 0)),
        ],
        out_specs=pl.BlockSpec((_TILE, _E), lambda i: (i, 0)),
        out_shape=jax.ShapeDtypeStruct((_N_TOKENS, _E), jnp.float32),
    )(x, wt, b2)

    mesh = plsc.VectorSubcoreMesh(core_axis_name="c", subcore_axis_name="s")
    val, arg = pl.kernel(
        _route_kernel,
        mesh=mesh,
        out_type=[
            jax.ShapeDtypeStruct((_N_TOKENS,), jnp.float32),
            jax.ShapeDtypeStruct((_N_TOKENS,), jnp.int32),
        ],
        scratch_types=[
            pltpu.VMEM((_ROWS_PER_W * _E,), jnp.float32),
            pltpu.VMEM((_ROWS_PER_W,), jnp.float32),
            pltpu.VMEM((_ROWS_PER_W,), jnp.int32),
        ],
        compiler_params=pltpu.CompilerParams(needs_layout_passes=False),
    )(logits.reshape(_N_TOKENS * _E))

    return pl.pallas_call(
        partial(_assemble_kernel, n_tiles=n_tiles, tile=_TILE,
                capacity=capacity),
        grid=(n_tiles,),
        in_specs=[
            pl.BlockSpec((_TILE, 1), lambda i: (i, 0)),
            pl.BlockSpec((_TILE, 1), lambda i: (i, 0)),
        ],
        out_specs=pl.BlockSpec((_N_TOKENS, _E), lambda i: (0, 0)),
        out_shape=jax.ShapeDtypeStruct((_N_TOKENS, _E), jnp.float32),
        scratch_shapes=[pltpu.VMEM((1, _E), jnp.float32)],
    )(val.reshape(_N_TOKENS, 1), arg.reshape(_N_TOKENS, 1))
